# Initial kernel scaffold; baseline (speedup 1.0000x reference)
import jax
import jax.numpy as jnp
from jax.experimental import pallas as pl


def kernel(x, M):
    raise NotImplementedError("write your pallas kernel here")


# speedup vs baseline: 3.5655x; 3.5655x over previous
import functools

import jax
import jax.numpy as jnp
from jax import lax
from jax.experimental import pallas as pl
from jax.experimental.pallas import tpu as pltpu
from jax.experimental.pallas import tpu_sc as plsc

NC, NS, L = 2, 16, 16
NW = NC * NS
B = 16384
V = 256
D = V * 3
T = 254
NG = (T + L - 1) // L
ROWS_PER_W = B // NW
CH = 32
NCHUNK = ROWS_PER_W // CH

INV_CBRT_MAGIC = 1420470954


def _sc_body(x_hbm, m_hbm, out_hbm, mbuf, mcol, xbuf, accbuf):
    wid = lax.axis_index("s") * NC + lax.axis_index("c")
    lane = lax.iota(jnp.int32, L)

    pltpu.sync_copy(m_hbm, mbuf)
    for g in range(NG):
        t = g * L + lane
        valid = t < T
        tc = jnp.where(valid, t, 0)
        for j in range(3):
            mj = plsc.load_gather(mbuf, [tc * 3 + j])
            for comp in range(3):
                col = jnp.where(valid, mj * 3 + comp, 0)
                mcol[pl.ds((g * 9 + j * 3 + comp) * L, L)] = col

    def chunk_body(ci, carry):
        base = (wid * ROWS_PER_W + ci * CH) * D
        pltpu.sync_copy(x_hbm.at[pl.ds(base, CH * D)], xbuf)

        for g in range(NG):
            cols = [mcol[pl.ds((g * 9 + s) * L, L)] for s in range(9)]
            gmask = (g * L + lane) < T

            def tri_body(r, carry, cols=cols, g=g, gmask=gmask):
                rb = r * D
                vals = [plsc.load_gather(xbuf, [c + rb]) for c in cols]
                ax, ay, az, bx, by, bz, cx, cy, cz = vals
                det = (ax * (by * cz - bz * cy)
                       - ay * (bx * cz - bz * cx)
                       + az * (bx * cy - by * cx))
                ad = jnp.abs(det)
                if g == NG - 1:
                    ad = jnp.where(gmask, ad, 0.0)
                if g == 0:
                    accbuf[pl.ds(r * L, L)] = ad
                else:
                    plsc.addupdate(accbuf.at[pl.ds(r * L, L)], ad)
                return carry

            lax.fori_loop(0, CH, tri_body, 0, unroll=2)

        def norm_body(r, carry):
            acc = accbuf[pl.ds(r * L, L)]
            vol = jnp.sum(acc) * (1.0 / 6.0)
            v = jnp.full((L,), vol, jnp.float32)
            bits = plsc.bitcast(v, jnp.int32)
            y = plsc.bitcast(INV_CBRT_MAGIC - lax.div(bits, jnp.int32(3)),
                             jnp.float32)
            for _ in range(3):
                y = y * (4.0 - v * y * y * y) * (1.0 / 3.0)
            rb = r * D
            for j in range(D // L):
                xbuf[pl.ds(rb + j * L, L)] = xbuf[pl.ds(rb + j * L, L)] * y
            return carry

        lax.fori_loop(0, CH, norm_body, 0)

        pltpu.sync_copy(xbuf, out_hbm.at[pl.ds(base, CH * D)])
        return carry

    lax.fori_loop(0, NCHUNK, chunk_body, 0)


_mesh = plsc.VectorSubcoreMesh(
    core_axis_name="c", subcore_axis_name="s", num_cores=NC, num_subcores=NS
)

_sc_call = functools.partial(
    pl.kernel,
    out_type=jax.ShapeDtypeStruct((B * D,), jnp.float32),
    mesh=_mesh,
    scratch_types=[
        pltpu.VMEM((T * 3 + 6,), jnp.int32),
        pltpu.VMEM((NG * 9 * L,), jnp.int32),
        pltpu.VMEM((CH * D,), jnp.float32),
        pltpu.VMEM((CH * L,), jnp.float32),
    ],
    compiler_params=pltpu.CompilerParams(needs_layout_passes=False),
)(_sc_body)


@jax.jit
def kernel(x, M):
    x1 = x.reshape(-1)
    m1 = jnp.concatenate([M.reshape(-1).astype(jnp.int32),
                          jnp.zeros((6,), jnp.int32)])
    out = _sc_call(x1, m1)
    return out.reshape(x.shape)


# speedup vs baseline: 4.9309x; 1.3829x over previous
import functools

import jax
import jax.numpy as jnp
from jax import lax
from jax.experimental import pallas as pl
from jax.experimental.pallas import tpu as pltpu
from jax.experimental.pallas import tpu_sc as plsc

NC, NS, L = 2, 16, 16
NW = NC * NS
B = 16384
V = 256
D = V * 3
T = 254
NG = (T + L - 1) // L
ROWS_PER_W = B // NW
CH = 32
NCHUNK = ROWS_PER_W // CH

INV_CBRT_MAGIC = 1420470954


def _sc_body(x_hbm, m_hbm, out_hbm, mbuf, mcol, xbuf, accbuf):
    wid = lax.axis_index("s") * NC + lax.axis_index("c")
    lane = lax.iota(jnp.int32, L)

    pltpu.sync_copy(m_hbm, mbuf)
    for g in range(NG):
        t = g * L + lane
        valid = t < T
        tc = jnp.where(valid, t, 0)
        for j in range(3):
            mj = plsc.load_gather(mbuf, [tc * 3 + j])
            for comp in range(3):
                col = jnp.where(valid, mj * 3 + comp, 0)
                mcol[pl.ds((g * 9 + j * 3 + comp) * L, L)] = col

    def chunk_body(ci, carry):
        base = (wid * ROWS_PER_W + ci * CH) * D
        pltpu.sync_copy(x_hbm.at[pl.ds(base, CH * D)], xbuf)

        for g in range(NG):
            cols = [mcol[pl.ds((g * 9 + s) * L, L)] for s in range(9)]
            gmask = (g * L + lane) < T

            @plsc.parallel_loop(0, CH, 1, unroll=2)
            def tri_body(r, cols=cols, g=g, gmask=gmask):
                rb = r * D
                vals = [plsc.load_gather(xbuf, [c + rb]) for c in cols]
                ax, ay, az, bx, by, bz, cx, cy, cz = vals
                det = (ax * (by * cz - bz * cy)
                       - ay * (bx * cz - bz * cx)
                       + az * (bx * cy - by * cx))
                ad = jnp.abs(det)
                if g == NG - 1:
                    ad = jnp.where(gmask, ad, 0.0)
                if g == 0:
                    accbuf[pl.ds(r * L, L)] = ad
                else:
                    plsc.addupdate(accbuf.at[pl.ds(r * L, L)], ad)

        @plsc.parallel_loop(0, CH, 1)
        def norm_body(r):
            acc = accbuf[pl.ds(r * L, L)]
            vol = jnp.sum(acc) * (1.0 / 6.0)
            v = jnp.full((L,), vol, jnp.float32)
            bits = plsc.bitcast(v, jnp.int32)
            y = plsc.bitcast(INV_CBRT_MAGIC - lax.div(bits, jnp.int32(3)),
                             jnp.float32)
            for _ in range(3):
                y = y * (4.0 - v * y * y * y) * (1.0 / 3.0)
            rb = r * D
            for j in range(D // L):
                xbuf[pl.ds(rb + j * L, L)] = xbuf[pl.ds(rb + j * L, L)] * y

        pltpu.sync_copy(xbuf, out_hbm.at[pl.ds(base, CH * D)])
        return carry

    lax.fori_loop(0, NCHUNK, chunk_body, 0)


_mesh = plsc.VectorSubcoreMesh(
    core_axis_name="c", subcore_axis_name="s", num_cores=NC, num_subcores=NS
)

_sc_call = functools.partial(
    pl.kernel,
    out_type=jax.ShapeDtypeStruct((B * D,), jnp.float32),
    mesh=_mesh,
    scratch_types=[
        pltpu.VMEM((T * 3 + 6,), jnp.int32),
        pltpu.VMEM((NG * 9 * L,), jnp.int32),
        pltpu.VMEM((CH * D,), jnp.float32),
        pltpu.VMEM((CH * L,), jnp.float32),
    ],
    compiler_params=pltpu.CompilerParams(needs_layout_passes=False),
)(_sc_body)


@jax.jit
def kernel(x, M):
    x1 = x.reshape(-1)
    m1 = jnp.concatenate([M.reshape(-1).astype(jnp.int32),
                          jnp.zeros((6,), jnp.int32)])
    out = _sc_call(x1, m1)
    return out.reshape(x.shape)
